# baseline (speedup 1.0000x reference)
import functools

import jax
import jax.numpy as jnp
from jax.experimental import pallas as pl
from jax.experimental.pallas import tpu as pltpu


def _fused_kernel(n_sel, idxs_ref,
                  x_ref, w1_ref, b1_ref, w2_ref, b2_ref,
                  wh_ref, bh_ref, wl_ref, bl_ref, o_ref):
    h = jnp.dot(x_ref[...], w1_ref[...], preferred_element_type=jnp.float32)
    h = jnp.maximum(h + b1_ref[...], 0.0).astype(jnp.bfloat16)
    f = jnp.dot(h, w2_ref[...], preferred_element_type=jnp.float32)
    f = jnp.maximum(f + b2_ref[...], 0.0).astype(jnp.bfloat16)

    for j in range(n_sel):
        idx = idxs_ref[j]
        hh = jnp.dot(f, wh_ref[idx], preferred_element_type=jnp.float32)
        hh = jnp.maximum(hh + bh_ref[idx], 0.0).astype(jnp.bfloat16)
        o_ref[j] = (jnp.dot(hh, wl_ref[idx], preferred_element_type=jnp.float32)
                    + bl_ref[idx])


def _forward(x, w1, b1, w2, b2, wh_all, bh_all, wl_all, bl_all, head_idxs):
    B, in_dim = x.shape
    out_dim = wl_all.shape[-1]
    n_sel = head_idxs.shape[0]

    rows = 1024 if B % 1024 == 0 else 512
    if B % rows != 0:
        b_pad = ((B + rows - 1) // rows) * rows
        x = jnp.pad(x, ((0, b_pad - B), (0, 0)))
    else:
        b_pad = B

    xb = x.astype(jnp.bfloat16)
    w1b = w1.astype(jnp.bfloat16)
    w2b = w2.astype(jnp.bfloat16)
    whb = wh_all.astype(jnp.bfloat16)
    wlb = wl_all.astype(jnp.bfloat16)
    idxs = head_idxs.astype(jnp.int32)

    grid_spec = pltpu.PrefetchScalarGridSpec(
        num_scalar_prefetch=1,
        grid=(b_pad // rows,),
        in_specs=[
            pl.BlockSpec((rows, in_dim), lambda i, idxs: (i, 0)),
            pl.BlockSpec(w1b.shape, lambda i, idxs: (0, 0)),
            pl.BlockSpec(b1.shape, lambda i, idxs: (0, 0)),
            pl.BlockSpec(w2b.shape, lambda i, idxs: (0, 0)),
            pl.BlockSpec(b2.shape, lambda i, idxs: (0, 0)),
            pl.BlockSpec(whb.shape, lambda i, idxs: (0, 0, 0)),
            pl.BlockSpec(bh_all.shape, lambda i, idxs: (0, 0, 0)),
            pl.BlockSpec(wlb.shape, lambda i, idxs: (0, 0, 0)),
            pl.BlockSpec(bl_all.shape, lambda i, idxs: (0, 0, 0)),
        ],
        out_specs=pl.BlockSpec((n_sel, rows, out_dim), lambda i, idxs: (0, i, 0)),
    )

    out = pl.pallas_call(
        functools.partial(_fused_kernel, n_sel),
        out_shape=jax.ShapeDtypeStruct((n_sel, b_pad, out_dim), jnp.float32),
        grid_spec=grid_spec,
        compiler_params=pltpu.CompilerParams(dimension_semantics=("parallel",)),
    )(idxs, xb, w1b, b1, w2b, b2, whb, bh_all, wlb, bl_all)

    return out[:, :B, :]


def kernel(x, w1, b1, w2, b2, wh_all, bh_all, wl_all, bl_all, head_idxs):
    return _forward(x, w1, b1, w2, b2, wh_all, bh_all, wl_all, bl_all,
                    head_idxs)


# speedup vs baseline: 1.3062x; 1.3062x over previous
import functools

import jax
import jax.numpy as jnp
from jax.experimental import pallas as pl
from jax.experimental.pallas import tpu as pltpu


def _fused_kernel(n_sel, idxs_ref,
                  x_ref, w1_ref, b1_ref, w2_ref, b2_ref,
                  wh_ref, bh_ref, wl_ref, bl_ref, o_ref):
    xb = x_ref[...].astype(jnp.bfloat16)
    h = jnp.dot(xb, w1_ref[...].astype(jnp.bfloat16),
                preferred_element_type=jnp.float32)
    h = jnp.maximum(h + b1_ref[...], 0.0).astype(jnp.bfloat16)
    f = jnp.dot(h, w2_ref[...].astype(jnp.bfloat16),
                preferred_element_type=jnp.float32)
    f = jnp.maximum(f + b2_ref[...], 0.0).astype(jnp.bfloat16)

    for j in range(n_sel):
        idx = idxs_ref[j]
        hh = jnp.dot(f, wh_ref[idx].astype(jnp.bfloat16),
                     preferred_element_type=jnp.float32)
        hh = jnp.maximum(hh + bh_ref[idx], 0.0).astype(jnp.bfloat16)
        o_ref[j] = (jnp.dot(hh, wl_ref[idx].astype(jnp.bfloat16),
                            preferred_element_type=jnp.float32)
                    + bl_ref[idx])


def _forward(x, w1, b1, w2, b2, wh_all, bh_all, wl_all, bl_all, head_idxs):
    B, in_dim = x.shape
    out_dim = wl_all.shape[-1]
    n_sel = head_idxs.shape[0]

    rows = 1024 if B % 1024 == 0 else 512
    if B % rows != 0:
        b_pad = ((B + rows - 1) // rows) * rows
        x = jnp.pad(x, ((0, b_pad - B), (0, 0)))
    else:
        b_pad = B

    idxs = head_idxs.astype(jnp.int32)

    grid_spec = pltpu.PrefetchScalarGridSpec(
        num_scalar_prefetch=1,
        grid=(b_pad // rows,),
        in_specs=[
            pl.BlockSpec((rows, in_dim), lambda i, idxs: (i, 0)),
            pl.BlockSpec(w1.shape, lambda i, idxs: (0, 0)),
            pl.BlockSpec(b1.shape, lambda i, idxs: (0, 0)),
            pl.BlockSpec(w2.shape, lambda i, idxs: (0, 0)),
            pl.BlockSpec(b2.shape, lambda i, idxs: (0, 0)),
            pl.BlockSpec(wh_all.shape, lambda i, idxs: (0, 0, 0)),
            pl.BlockSpec(bh_all.shape, lambda i, idxs: (0, 0, 0)),
            pl.BlockSpec(wl_all.shape, lambda i, idxs: (0, 0, 0)),
            pl.BlockSpec(bl_all.shape, lambda i, idxs: (0, 0, 0)),
        ],
        out_specs=pl.BlockSpec((n_sel, rows, out_dim), lambda i, idxs: (0, i, 0)),
    )

    out = pl.pallas_call(
        functools.partial(_fused_kernel, n_sel),
        out_shape=jax.ShapeDtypeStruct((n_sel, b_pad, out_dim), jnp.float32),
        grid_spec=grid_spec,
        compiler_params=pltpu.CompilerParams(
            dimension_semantics=("arbitrary",)),
    )(idxs, x, w1, b1, w2, b2, wh_all, bh_all, wl_all, bl_all)

    return out[:, :B, :]


def kernel(x, w1, b1, w2, b2, wh_all, bh_all, wl_all, bl_all, head_idxs):
    return _forward(x, w1, b1, w2, b2, wh_all, bh_all, wl_all, bl_all,
                    head_idxs)
